# Initial kernel scaffold; baseline (speedup 1.0000x reference)
import jax
import jax.numpy as jnp
from jax.experimental import pallas as pl


def kernel(x, edge_index, edge_label_index, W0, b0, W1, b1, W2, b2, Wp, bp):
    raise NotImplementedError("write your pallas kernel here")


# speedup vs baseline: 7.3764x; 7.3764x over previous
import functools

import jax
import jax.numpy as jnp
from jax import lax
from jax.experimental import pallas as pl
from jax.experimental.pallas import tpu as pltpu
from jax.experimental.pallas import tpu_sc as plsc

NPAD = 10240
NC = 2
NS = 16


def _sc_mesh():
    return plsc.VectorSubcoreMesh(
        core_axis_name="c", subcore_axis_name="s", num_cores=NC, num_subcores=NS
    )


def _fill(ref, length, value):
    def body(i, _):
        ref[pl.ds(i * 16, 16)] = jnp.full((16,), value, jnp.float32)
        return 0
    lax.fori_loop(0, length // 16, body, 0)


def _sc_deg(dst):
    E = dst.shape[0]
    ept = E // NS
    K = 400
    nw = ept // K
    rows = NPAD // NS

    @functools.partial(
        pl.kernel,
        out_type=jax.ShapeDtypeStruct((NPAD,), jnp.float32),
        mesh=_sc_mesh(),
        scratch_types=[
            pltpu.VMEM((K,), jnp.int32),
            pltpu.VMEM((K,), jnp.float32),
            pltpu.VMEM((rows,), jnp.float32),
            pltpu.VMEM_SHARED((NPAD,), jnp.float32),
        ],
    )
    def deg_kernel(dst_hbm, deg_hbm, idx_v, ones_v, row_v, acc_sh):
        cid = lax.axis_index("c")
        sid = lax.axis_index("s")
        _fill(ones_v, K, 1.0)
        _fill(row_v, rows, 1.0)

        @pl.when(cid == 0)
        def _():
            pltpu.sync_copy(row_v, acc_sh.at[pl.ds(sid * rows, rows)])
        plsc.subcore_barrier()

        @pl.when(cid == 0)
        def _():
            base = sid * ept
            def win(i, _):
                pltpu.sync_copy(dst_hbm.at[pl.ds(base + i * K, K)], idx_v)
                pltpu.sync_copy(ones_v, acc_sh.at[idx_v], add=True)
                return 0
            lax.fori_loop(0, nw, win, 0)
        plsc.subcore_barrier()

        @pl.when(cid == 0)
        def _():
            pltpu.sync_copy(acc_sh.at[pl.ds(sid * rows, rows)],
                            deg_hbm.at[pl.ds(sid * rows, rows)])

    return deg_kernel(dst)


def _sc_agg(hws, src, dst):
    E = src.shape[0]
    ept = E // NS
    K = 200
    nw = ept // K
    rows = NPAD // NS

    @functools.partial(
        pl.kernel,
        out_type=jax.ShapeDtypeStruct((4, NPAD, 128), jnp.float32),
        mesh=_sc_mesh(),
        scratch_types=[
            pltpu.VMEM((K,), jnp.int32),
            pltpu.VMEM((K,), jnp.int32),
            pltpu.VMEM((K, 128), jnp.float32),
            pltpu.VMEM_SHARED((NPAD, 128), jnp.float32),
            pltpu.SemaphoreType.DMA,
        ],
    )
    def agg_kernel(hws_hbm, src_hbm, dst_hbm, agg_hbm, si_v, di_v, rows_v,
                   acc_sh, sem):
        cid = lax.axis_index("c")
        sid = lax.axis_index("s")
        for phase in range(2):
            chunk = cid * 2 + phase
            pltpu.sync_copy(hws_hbm.at[chunk, pl.ds(sid * rows, rows)],
                            acc_sh.at[pl.ds(sid * rows, rows)])
            plsc.subcore_barrier()
            base = sid * ept
            def win(i, _):
                pltpu.sync_copy(src_hbm.at[pl.ds(base + i * K, K)], si_v)
                pltpu.sync_copy(dst_hbm.at[pl.ds(base + i * K, K)], di_v)
                pltpu.async_copy(hws_hbm.at[chunk].at[si_v], rows_v, sem).wait()
                pltpu.sync_copy(rows_v, acc_sh.at[di_v], add=True)
                return 0
            lax.fori_loop(0, nw, win, 0)
            plsc.subcore_barrier()
            pltpu.sync_copy(acc_sh.at[pl.ds(sid * rows, rows)],
                            agg_hbm.at[chunk, pl.ds(sid * rows, rows)])
            plsc.subcore_barrier()

    return agg_kernel(hws, src, dst)


def _sc_head(emb, g, sidx, didx, bsum16):
    EL = sidx.shape[0]
    K = 80
    nwin = EL // K
    rounds = (nwin + NC * NS - 1) // (NC * NS)

    @functools.partial(
        pl.kernel,
        out_type=jax.ShapeDtypeStruct((EL,), jnp.float32),
        mesh=_sc_mesh(),
        scratch_types=[
            pltpu.VMEM((K,), jnp.int32),
            pltpu.VMEM((K,), jnp.int32),
            pltpu.VMEM((K, 512), jnp.float32),
            pltpu.VMEM((K, 512), jnp.float32),
            pltpu.VMEM((K,), jnp.float32),
            pltpu.VMEM((256,), jnp.float32),
            pltpu.VMEM((16,), jnp.float32),
            pltpu.SemaphoreType.DMA,
            pltpu.SemaphoreType.DMA,
        ],
        compiler_params=pltpu.CompilerParams(needs_layout_passes=False),
    )
    def head_kernel(emb_hbm, g_hbm, sidx_hbm, didx_hbm, bsum_hbm, out_hbm,
                    si_v, di_v, a_v, b_v, sc_v, t_v, bs_v, sem1, sem2):
        cid = lax.axis_index("c")
        sid = lax.axis_index("s")
        wid = sid * NC + cid
        pltpu.sync_copy(bsum_hbm.at[0], bs_v)
        lanes = lax.iota(jnp.int32, 16)

        def win(j, _):
            w = wid + j * (NC * NS)

            @pl.when(w < nwin)
            def _():
                off = w * K
                pltpu.sync_copy(sidx_hbm.at[pl.ds(off, K)], si_v)
                pltpu.sync_copy(didx_hbm.at[pl.ds(off, K)], di_v)
                cp_a = pltpu.async_copy(emb_hbm.at[si_v], a_v, sem1)
                cp_b = pltpu.async_copy(g_hbm.at[di_v], b_v, sem2)
                cp_a.wait()
                cp_b.wait()

                def grp(gi, _):
                    def pair(p, _):
                        row = jnp.full((16,), gi * 16 + p, jnp.int32)
                        acc = jnp.zeros((16,), jnp.float32)
                        for c in range(32):
                            col = lanes + c * 16
                            av = plsc.load_gather(a_v, [row, col])
                            bv = plsc.load_gather(b_v, [row, col])
                            acc = acc + av * bv
                        t_v[pl.ds(p * 16, 16)] = acc
                        return 0
                    lax.fori_loop(0, 16, pair, 0)
                    s = bs_v[...]
                    for c in range(16):
                        idxv = lanes * 16 + c
                        s = s + plsc.load_gather(t_v, [idxv])
                    sc_v[pl.ds(gi * 16, 16)] = s
                    return 0
                lax.fori_loop(0, K // 16, grp, 0)
                pltpu.sync_copy(sc_v, out_hbm.at[pl.ds(off, K)])
            return 0
        lax.fori_loop(0, rounds, win, 0)

    return head_kernel(emb, g, sidx, didx, bsum16)


def _tc_layer0(xp, w0, deg2d):
    R = 512
    grid = NPAD // R
    d_in = xp.shape[1]
    h = w0.shape[1]

    def body(x_ref, w_ref, deg_ref, out_ref, dv_ref):
        dv = lax.rsqrt(deg_ref[...])
        dv_ref[...] = dv
        hw = jnp.dot(x_ref[...], w_ref[...], preferred_element_type=jnp.float32)
        hws = hw * dv
        for c in range(4):
            out_ref[c] = hws[:, c * 128:(c + 1) * 128]

    return pl.pallas_call(
        body,
        grid=(grid,),
        in_specs=[
            pl.BlockSpec((R, d_in), lambda i: (i, 0)),
            pl.BlockSpec((d_in, h), lambda i: (0, 0)),
            pl.BlockSpec((R, 1), lambda i: (i, 0)),
        ],
        out_specs=[
            pl.BlockSpec((4, R, 128), lambda i: (0, i, 0)),
            pl.BlockSpec((R, 1), lambda i: (i, 0)),
        ],
        out_shape=[
            jax.ShapeDtypeStruct((4, NPAD, 128), jnp.float32),
            jax.ShapeDtypeStruct((NPAD, 1), jnp.float32),
        ],
    )(xp, w0, deg2d)


def _tc_layer(n, agg, dinv2d, b2d, w):
    R = 512
    grid = NPAD // R
    h = w.shape[1]

    def body(a_ref, dv_ref, b_ref, w_ref, emb_ref, out_ref):
        a = jnp.concatenate([a_ref[c] for c in range(4)], axis=1)
        hact = jnp.maximum(a * dv_ref[...] + b_ref[...], 0.0)
        emb_ref[...] = hact
        hw = jnp.dot(hact, w_ref[...], preferred_element_type=jnp.float32)
        hws = hw * dv_ref[...]
        for c in range(4):
            out_ref[c] = hws[:, c * 128:(c + 1) * 128]

    return pl.pallas_call(
        body,
        grid=(grid,),
        in_specs=[
            pl.BlockSpec((4, R, 128), lambda i: (0, i, 0)),
            pl.BlockSpec((R, 1), lambda i: (i, 0)),
            pl.BlockSpec((1, h), lambda i: (0, 0)),
            pl.BlockSpec((h, h), lambda i: (0, 0)),
        ],
        out_specs=[
            pl.BlockSpec((R, h), lambda i: (i, 0)),
            pl.BlockSpec((4, R, 128), lambda i: (0, i, 0)),
        ],
        out_shape=[
            jax.ShapeDtypeStruct((n, h), jnp.float32),
            jax.ShapeDtypeStruct((4, NPAD, 128), jnp.float32),
        ],
    )(agg, dinv2d, b2d, w)


def _tc_final(n, agg, dinv2d, b2d, wpt, bp2d):
    R = 512
    grid = NPAD // R
    h = agg.shape[2] * 4

    def body(a_ref, dv_ref, b_ref, wp_ref, bp_ref, emb_ref, g_ref, bs_ref):
        a = jnp.concatenate([a_ref[c] for c in range(4)], axis=1)
        hact = jnp.maximum(a * dv_ref[...] + b_ref[...], 0.0)
        emb_ref[...] = hact
        wsum = jnp.sum(wp_ref[...], axis=0)[None, :]
        g_ref[...] = hact * wsum

        @pl.when(pl.program_id(0) == 0)
        def _():
            bs_ref[...] = jnp.full((1, 16), jnp.sum(bp_ref[...]), jnp.float32)

    return pl.pallas_call(
        body,
        grid=(grid,),
        in_specs=[
            pl.BlockSpec((4, R, 128), lambda i: (0, i, 0)),
            pl.BlockSpec((R, 1), lambda i: (i, 0)),
            pl.BlockSpec((1, h), lambda i: (0, 0)),
            pl.BlockSpec((2, h), lambda i: (0, 0)),
            pl.BlockSpec((1, 2), lambda i: (0, 0)),
        ],
        out_specs=[
            pl.BlockSpec((R, h), lambda i: (i, 0)),
            pl.BlockSpec((R, h), lambda i: (i, 0)),
            pl.BlockSpec((1, 16), lambda i: (0, 0)),
        ],
        out_shape=[
            jax.ShapeDtypeStruct((n, h), jnp.float32),
            jax.ShapeDtypeStruct((n, h), jnp.float32),
            jax.ShapeDtypeStruct((1, 16), jnp.float32),
        ],
    )(agg, dinv2d, b2d, wpt, bp2d)


def kernel(x, edge_index, edge_label_index, W0, b0, W1, b1, W2, b2, Wp, bp):
    n = x.shape[0]
    src = edge_index[0]
    dst = edge_index[1]
    sidx = edge_label_index[0]
    didx = edge_label_index[1]

    xp = jnp.pad(x, ((0, NPAD - n), (0, 0)))
    deg = _sc_deg(dst)

    hws0, dinv2d = _tc_layer0(xp, W0, deg.reshape(NPAD, 1))
    agg0 = _sc_agg(hws0, src, dst)
    emb0, hws1 = _tc_layer(n, agg0, dinv2d, b0.reshape(1, -1), W1)
    agg1 = _sc_agg(hws1, src, dst)
    emb1, hws2 = _tc_layer(n, agg1, dinv2d, b1.reshape(1, -1), W2)
    agg2 = _sc_agg(hws2, src, dst)
    emb2, g, bsum16 = _tc_final(n, agg2, dinv2d, b2.reshape(1, -1), Wp.T,
                                bp.reshape(1, 2))
    score = _sc_head(emb2, g, sidx, didx, bsum16)
    return (score, emb0, emb1, emb2)


# speedup vs baseline: 11.1343x; 1.5094x over previous
import functools

import jax
import jax.numpy as jnp
from jax import lax
from jax.experimental import pallas as pl
from jax.experimental.pallas import tpu as pltpu
from jax.experimental.pallas import tpu_sc as plsc

NROWS = 10000
NACC = 10112
NPAD = 10240
NC = 2
NS = 16


def _sc_mesh():
    return plsc.VectorSubcoreMesh(
        core_axis_name="c", subcore_axis_name="s", num_cores=NC, num_subcores=NS
    )


def _dot_ref(a, b):
    ah = a.astype(jnp.bfloat16)
    al = (a - ah.astype(jnp.float32)).astype(jnp.bfloat16)
    bh = b.astype(jnp.bfloat16)
    bl = (b - bh.astype(jnp.float32)).astype(jnp.bfloat16)
    kw = dict(preferred_element_type=jnp.float32)
    return ((jnp.dot(ah, bl, **kw) + jnp.dot(al, bh, **kw))
            + jnp.dot(ah, bh, **kw))


def _fill(ref, length, value):
    def body(i, _):
        ref[pl.ds(i * 16, 16)] = jnp.full((16,), value, jnp.float32)
        return 0
    lax.fori_loop(0, length // 16, body, 0)


def _sc_deg(dst):
    E = dst.shape[0]
    ept = E // NS
    K = 336
    nw = ept // K
    rows = NPAD // NS

    @functools.partial(
        pl.kernel,
        out_type=jax.ShapeDtypeStruct((NPAD,), jnp.float32),
        mesh=_sc_mesh(),
        scratch_types=[
            pltpu.VMEM((K,), jnp.int32),
            pltpu.VMEM((K,), jnp.float32),
            pltpu.VMEM((rows,), jnp.float32),
            pltpu.VMEM_SHARED((NPAD,), jnp.float32),
        ],
    )
    def deg_kernel(dst_hbm, deg_hbm, idx_v, ones_v, row_v, acc_sh):
        cid = lax.axis_index("c")
        sid = lax.axis_index("s")
        _fill(ones_v, K, 1.0)
        _fill(row_v, rows, 1.0)

        @pl.when(cid == 0)
        def _():
            pltpu.sync_copy(row_v, acc_sh.at[pl.ds(sid * rows, rows)])
        plsc.subcore_barrier()

        @pl.when(cid == 0)
        def _():
            base = sid * ept
            def win(i, _):
                pltpu.sync_copy(dst_hbm.at[pl.ds(base + i * K, K)], idx_v)
                pltpu.sync_copy(ones_v, acc_sh.at[idx_v], add=True)
                return 0
            lax.fori_loop(0, nw, win, 0)
        plsc.subcore_barrier()

        @pl.when(cid == 0)
        def _():
            pltpu.sync_copy(acc_sh.at[pl.ds(sid * rows, rows)],
                            deg_hbm.at[pl.ds(sid * rows, rows)])

    return deg_kernel(dst)


def _sc_agg(n, hws, src3, dst_flat):
    NS_, nw, K = src3.shape
    ept = nw * K
    nacc = NACC
    rows = nacc // NS

    @functools.partial(
        pl.kernel,
        out_type=jax.ShapeDtypeStruct((4, nacc, 128), jnp.float32),
        mesh=_sc_mesh(),
        scratch_types=[
            pltpu.VMEM((nw, K), jnp.int32),
            pltpu.VMEM((K,), jnp.int32),
            pltpu.VMEM((K,), jnp.int32),
            pltpu.VMEM((K, 128), jnp.float32),
            pltpu.VMEM((K, 128), jnp.float32),
            pltpu.VMEM_SHARED((nacc, 128), jnp.float32),
            pltpu.SemaphoreType.DMA,
            pltpu.SemaphoreType.DMA,
            pltpu.SemaphoreType.DMA,
            pltpu.SemaphoreType.DMA,
            pltpu.SemaphoreType.DMA,
            pltpu.SemaphoreType.DMA,
        ],
        compiler_params=pltpu.CompilerParams(needs_layout_passes=False),
    )
    def agg_kernel(hws_hbm, src_hbm, dstf_hbm, agg_hbm, si_v, di0, di1,
                   rows0, rows1, acc_sh, g0, g1, s0, s1, d0, d1):
        cid = lax.axis_index("c")
        sid = lax.axis_index("s")
        di_b = (di0, di1)
        rows_b = (rows0, rows1)
        gsem = (g0, g1)
        ssem = (s0, s1)
        dsem = (d0, d1)
        base = sid * ept
        pltpu.sync_copy(src_hbm.at[sid], si_v)
        for phase in range(2):
            chunk = cid * 2 + phase
            pltpu.sync_copy(hws_hbm.at[chunk, pl.ds(sid * rows, rows)],
                            acc_sh.at[pl.ds(sid * rows, rows)])
            plsc.subcore_barrier()

            def start_window(i, b):
                pltpu.async_copy(dstf_hbm.at[pl.ds(base + i * K, K)],
                                 di_b[b], dsem[b])
                pltpu.async_copy(hws_hbm.at[chunk].at[si_v.at[i]],
                                 rows_b[b], gsem[b])

            def wait_window(i, b):
                pltpu.make_async_copy(dstf_hbm.at[pl.ds(base + i * K, K)],
                                      di_b[b], dsem[b]).wait()
                pltpu.make_async_copy(hws_hbm.at[chunk].at[si_v.at[i]],
                                      rows_b[b], gsem[b]).wait()

            def start_scatter(i, b):
                pltpu.async_copy(rows_b[b], acc_sh.at[di_b[b]],
                                 ssem[b], add=True)

            def wait_scatter(b):
                pltpu.make_async_copy(rows_b[b], acc_sh.at[di_b[b]],
                                      ssem[b]).wait()

            start_window(0, 0)

            def win2(i2, carry):
                for half in range(2):
                    i = i2 * 2 + half
                    b = half
                    nb = 1 - half

                    @pl.when(i + 1 < nw)
                    def _():
                        @pl.when(i >= 1)
                        def _():
                            wait_scatter(nb)
                        start_window(i + 1, nb)
                    wait_window(i, b)
                    start_scatter(i, b)
                return carry
            lax.fori_loop(0, nw // 2, win2, 0)
            wait_scatter(0)
            wait_scatter(1)
            plsc.subcore_barrier()
            pltpu.sync_copy(acc_sh.at[pl.ds(sid * rows, rows)],
                            agg_hbm.at[chunk, pl.ds(sid * rows, rows)])
            plsc.subcore_barrier()

    return agg_kernel(hws, src3, dst_flat)


def _sc_head(emb, g, sidx, didx, bsum16):
    elp = sidx.shape[0]
    K = 48
    rounds = elp // (K * NC * NS)

    @functools.partial(
        pl.kernel,
        out_type=jax.ShapeDtypeStruct((elp,), jnp.float32),
        mesh=_sc_mesh(),
        scratch_types=[
            pltpu.VMEM((K,), jnp.int32),
            pltpu.VMEM((K,), jnp.int32),
            pltpu.VMEM((K,), jnp.int32),
            pltpu.VMEM((K,), jnp.int32),
            pltpu.VMEM((K, 512), jnp.float32),
            pltpu.VMEM((K, 512), jnp.float32),
            pltpu.VMEM((K, 512), jnp.float32),
            pltpu.VMEM((K, 512), jnp.float32),
            pltpu.VMEM((K,), jnp.float32),
            pltpu.VMEM((256,), jnp.float32),
            pltpu.VMEM((16,), jnp.float32),
            pltpu.SemaphoreType.DMA,
            pltpu.SemaphoreType.DMA,
            pltpu.SemaphoreType.DMA,
            pltpu.SemaphoreType.DMA,
        ],
        compiler_params=pltpu.CompilerParams(needs_layout_passes=False),
    )
    def head_kernel(emb_hbm, g_hbm, sidx_hbm, didx_hbm, bsum_hbm, out_hbm,
                    si0, si1, di0, di1, a0, a1, b0, b1,
                    sc_v, t_v, bs_v, sa0, sa1, sb0, sb1):
        cid = lax.axis_index("c")
        sid = lax.axis_index("s")
        wid = sid * NC + cid
        si_b = (si0, si1)
        di_b = (di0, di1)
        a_b = (a0, a1)
        b_b = (b0, b1)
        sema = (sa0, sa1)
        semb = (sb0, sb1)
        pltpu.sync_copy(bsum_hbm.at[0], bs_v)
        lanes = lax.iota(jnp.int32, 16)

        def start_gather(j, p):
            off = (wid + j * (NC * NS)) * K
            pltpu.sync_copy(sidx_hbm.at[pl.ds(off, K)], si_b[p])
            pltpu.sync_copy(didx_hbm.at[pl.ds(off, K)], di_b[p])
            pltpu.async_copy(emb_hbm.at[si_b[p]], a_b[p], sema[p])
            pltpu.async_copy(g_hbm.at[di_b[p]], b_b[p], semb[p])

        def wait_gather(p):
            pltpu.make_async_copy(emb_hbm.at[si_b[p]], a_b[p], sema[p]).wait()
            pltpu.make_async_copy(g_hbm.at[di_b[p]], b_b[p], semb[p]).wait()

        def compute(j, p):
            a_v = a_b[p]
            b_v = b_b[p]
            off = (wid + j * (NC * NS)) * K

            def pair(pi, _):
                row = jnp.full((16,), pi, jnp.int32)
                acc = jnp.zeros((16,), jnp.float32)
                for c in range(32):
                    col = lanes + c * 16
                    av = plsc.load_gather(a_v, [row, col])
                    bv = plsc.load_gather(b_v, [row, col])
                    acc = acc + av * bv
                t_v[pl.ds((pi % 16) * 16, 16)] = acc

                @pl.when(pi % 16 == 15)
                def _():
                    s = bs_v[...]
                    for c in range(16):
                        s = s + plsc.load_gather(t_v, [lanes * 16 + c])
                    sc_v[pl.ds((pi // 16) * 16, 16)] = s
                return 0
            lax.fori_loop(0, K, pair, 0)
            pltpu.sync_copy(sc_v, out_hbm.at[pl.ds(off, K)])

        start_gather(0, 0)

        def round2(j2, carry):
            for half in range(2):
                j = j2 * 2 + half
                p = half
                np_ = 1 - half

                @pl.when(j + 1 < rounds)
                def _():
                    start_gather(j + 1, np_)
                wait_gather(p)
                compute(j, p)
            return carry
        lax.fori_loop(0, rounds // 2, round2, 0)

    return head_kernel(emb, g, sidx, didx, bsum16)


def _tc_layer0(xp, w0, deg2d):
    R = 512
    grid = NPAD // R
    d_in = xp.shape[1]
    h = w0.shape[1]

    def body(x_ref, w_ref, deg_ref, out_ref, dv_ref):
        dv = lax.rsqrt(deg_ref[...])
        dv_ref[...] = dv
        hw = _dot_ref(x_ref[...], w_ref[...])
        rid = (jax.lax.broadcasted_iota(jnp.int32, (R, 1), 0)
               + pl.program_id(0) * R)
        hws = jnp.where(rid < NROWS, hw * dv, 0.0)
        for c in range(4):
            out_ref[c] = hws[:, c * 128:(c + 1) * 128]

    return pl.pallas_call(
        body,
        grid=(grid,),
        in_specs=[
            pl.BlockSpec((R, d_in), lambda i: (i, 0)),
            pl.BlockSpec((d_in, h), lambda i: (0, 0)),
            pl.BlockSpec((R, 1), lambda i: (i, 0)),
        ],
        out_specs=[
            pl.BlockSpec((4, R, 128), lambda i: (0, i, 0)),
            pl.BlockSpec((R, 1), lambda i: (i, 0)),
        ],
        out_shape=[
            jax.ShapeDtypeStruct((4, NPAD, 128), jnp.float32),
            jax.ShapeDtypeStruct((NPAD, 1), jnp.float32),
        ],
    )(xp, w0, deg2d)


def _tc_layer(n, agg, dinv2d, b2d, w):
    R = 512
    grid = NPAD // R
    h = w.shape[1]

    def body(a_ref, dv_ref, b_ref, w_ref, emb_ref, out_ref):
        a = jnp.concatenate([a_ref[c] for c in range(4)], axis=1)
        hact = jnp.maximum(a * dv_ref[...] + b_ref[...], 0.0)
        emb_ref[...] = hact
        hw = _dot_ref(hact, w_ref[...])
        rid = (jax.lax.broadcasted_iota(jnp.int32, (R, 1), 0)
               + pl.program_id(0) * R)
        hws = jnp.where(rid < NROWS, hw * dv_ref[...], 0.0)
        for c in range(4):
            out_ref[c] = hws[:, c * 128:(c + 1) * 128]

    return pl.pallas_call(
        body,
        grid=(grid,),
        in_specs=[
            pl.BlockSpec((4, R, 128), lambda i: (0, i, 0)),
            pl.BlockSpec((R, 1), lambda i: (i, 0)),
            pl.BlockSpec((1, h), lambda i: (0, 0)),
            pl.BlockSpec((h, h), lambda i: (0, 0)),
        ],
        out_specs=[
            pl.BlockSpec((R, h), lambda i: (i, 0)),
            pl.BlockSpec((4, R, 128), lambda i: (0, i, 0)),
        ],
        out_shape=[
            jax.ShapeDtypeStruct((n, h), jnp.float32),
            jax.ShapeDtypeStruct((4, NPAD, 128), jnp.float32),
        ],
    )(agg, dinv2d, b2d, w)


def _tc_final(n, agg, dinv2d, b2d, wpt, bp2d):
    R = 512
    grid = NPAD // R
    h = agg.shape[2] * 4

    def body(a_ref, dv_ref, b_ref, wp_ref, bp_ref, emb_ref, g_ref, bs_ref):
        a = jnp.concatenate([a_ref[c] for c in range(4)], axis=1)
        hact = jnp.maximum(a * dv_ref[...] + b_ref[...], 0.0)
        emb_ref[...] = hact
        wsum = jnp.sum(wp_ref[...], axis=0)[None, :]
        g_ref[...] = hact * wsum

        @pl.when(pl.program_id(0) == 0)
        def _():
            bs_ref[...] = jnp.full((1, 16), jnp.sum(bp_ref[...]), jnp.float32)

    return pl.pallas_call(
        body,
        grid=(grid,),
        in_specs=[
            pl.BlockSpec((4, R, 128), lambda i: (0, i, 0)),
            pl.BlockSpec((R, 1), lambda i: (i, 0)),
            pl.BlockSpec((1, h), lambda i: (0, 0)),
            pl.BlockSpec((2, h), lambda i: (0, 0)),
            pl.BlockSpec((1, 2), lambda i: (0, 0)),
        ],
        out_specs=[
            pl.BlockSpec((R, h), lambda i: (i, 0)),
            pl.BlockSpec((R, h), lambda i: (i, 0)),
            pl.BlockSpec((1, 16), lambda i: (0, 0)),
        ],
        out_shape=[
            jax.ShapeDtypeStruct((n, h), jnp.float32),
            jax.ShapeDtypeStruct((n, h), jnp.float32),
            jax.ShapeDtypeStruct((1, 16), jnp.float32),
        ],
    )(agg, dinv2d, b2d, wpt, bp2d)


def kernel(x, edge_index, edge_label_index, W0, b0, W1, b1, W2, b2, Wp, bp):
    n = x.shape[0]
    e = edge_index.shape[1]
    kagg, nwagg = 112, 90
    epad = NS * kagg * nwagg
    npd = epad - e
    pad_zero_rows = n + jnp.arange(npd, dtype=jnp.int32) % (NPAD - n)
    src = jnp.concatenate([edge_index[0], pad_zero_rows])
    dst_agg = jnp.concatenate(
        [edge_index[1], jnp.arange(npd, dtype=jnp.int32) % n])
    dst_deg = jnp.concatenate([edge_index[1], pad_zero_rows])
    src3 = src.reshape(NS, nwagg, kagg)
    el = edge_label_index.shape[1]
    wpr = 48 * NC * NS
    elp = ((el + 2 * wpr - 1) // (2 * wpr)) * (2 * wpr)
    pad_pair = jnp.arange(elp - el, dtype=jnp.int32) % n
    sidx = jnp.concatenate([edge_label_index[0], pad_pair])
    didx = jnp.concatenate([edge_label_index[1], pad_pair])

    xp = jnp.pad(x, ((0, NPAD - n), (0, 0)))
    deg = _sc_deg(dst_deg)

    hws0, dinv2d = _tc_layer0(xp, W0, deg.reshape(NPAD, 1))
    agg0 = _sc_agg(n, hws0, src3, dst_agg)
    emb0, hws1 = _tc_layer(n, agg0, dinv2d, b0.reshape(1, -1), W1)
    agg1 = _sc_agg(n, hws1, src3, dst_agg)
    emb1, hws2 = _tc_layer(n, agg1, dinv2d, b1.reshape(1, -1), W2)
    agg2 = _sc_agg(n, hws2, src3, dst_agg)
    emb2, g, bsum16 = _tc_final(n, agg2, dinv2d, b2.reshape(1, -1), Wp.T,
                                bp.reshape(1, 2))
    score = _sc_head(emb2, g, sidx, didx, bsum16)[:el]
    return (score, emb0, emb1, emb2)
